# baseline (speedup 1.0000x reference)
import jax
import jax.numpy as jnp
from jax.experimental import pallas as pl
from jax.experimental.pallas import tpu as pltpu

_BBLK = 8


def _head_kernel(xs_hbm, g_ref, wt_ref, b_ref, o_ref, xbuf, sems):
    t = pl.program_id(0)
    nt = pl.num_programs(0)
    ll = xs_hbm.shape[3]
    ps = xs_hbm.shape[0]
    bblk = xbuf.shape[2]

    def copy(slot, tt, i):
        return pltpu.make_async_copy(
            xs_hbm.at[i, pl.ds(tt * bblk, bblk), :, ll - 1, :],
            xbuf.at[slot, i],
            sems.at[slot, i],
        )

    @pl.when(t == 0)
    def _():
        for i in range(ps):
            copy(0, 0, i).start()

    @pl.when(t + 1 < nt)
    def _():
        for i in range(ps):
            copy((t + 1) % 2, t + 1, i).start()

    for i in range(ps):
        copy(t % 2, t, i).wait()
    x = xbuf[t % 2]

    g = jnp.maximum(g_ref[...], 0.0)
    comb = x[0] * g[:, 0][:, None, None]
    for i in range(1, ps):
        comb = comb + x[i] * g[:, i][:, None, None]
    comb = comb + 1e-9
    res = jax.lax.dot_general(
        comb, wt_ref[...], (((2,), (0,)), ((), ())),
        preferred_element_type=jnp.float32)
    res = res + b_ref[0][None, None, :]
    o_ref[...] = jnp.transpose(res, (0, 2, 1))


def kernel(xs, gates, W, b):
    ps, bb, cc, ll, dd = xs.shape
    pred = W.shape[0]
    wt = W.T
    b2 = b.reshape(1, pred)
    grid = (bb // _BBLK,)
    return pl.pallas_call(
        _head_kernel,
        grid=grid,
        in_specs=[
            pl.BlockSpec(memory_space=pl.ANY),
            pl.BlockSpec((_BBLK, ps), lambda t: (t, 0)),
            pl.BlockSpec((dd, pred), lambda t: (0, 0)),
            pl.BlockSpec((1, pred), lambda t: (0, 0)),
        ],
        out_specs=pl.BlockSpec((_BBLK, pred, cc), lambda t: (t, 0, 0)),
        out_shape=jax.ShapeDtypeStruct((bb, pred, cc), jnp.float32),
        scratch_shapes=[
            pltpu.VMEM((2, ps, _BBLK, cc, dd), jnp.float32),
            pltpu.SemaphoreType.DMA((2, ps)),
        ],
    )(xs, gates, wt, b2)


# speedup vs baseline: 1.0679x; 1.0679x over previous
import jax
import jax.numpy as jnp
from jax.experimental import pallas as pl
from jax.experimental.pallas import tpu as pltpu

_BBLK = 16


def _head_kernel(xs_hbm, g_ref, wt_ref, b_ref, o_ref, xbuf, sems):
    t = pl.program_id(0)
    nt = pl.num_programs(0)
    ll = xs_hbm.shape[3]
    ps = xs_hbm.shape[0]
    bblk = xbuf.shape[2]

    def copy(slot, tt, i):
        return pltpu.make_async_copy(
            xs_hbm.at[i, pl.ds(tt * bblk, bblk), :, ll - 1, :],
            xbuf.at[slot, i],
            sems.at[slot, i],
        )

    @pl.when(t == 0)
    def _():
        for i in range(ps):
            copy(0, 0, i).start()

    @pl.when(t + 1 < nt)
    def _():
        for i in range(ps):
            copy((t + 1) % 2, t + 1, i).start()

    for i in range(ps):
        copy(t % 2, t, i).wait()
    x = xbuf[t % 2]

    g = jnp.maximum(g_ref[...], 0.0)
    comb = x[0] * g[:, 0][:, None, None]
    for i in range(1, ps):
        comb = comb + x[i] * g[:, i][:, None, None]
    comb = comb + 1e-9
    res = jax.lax.dot_general(
        comb, wt_ref[...], (((2,), (0,)), ((), ())),
        preferred_element_type=jnp.float32)
    res = res + b_ref[0][None, None, :]
    o_ref[...] = jnp.transpose(res, (0, 2, 1))


def kernel(xs, gates, W, b):
    ps, bb, cc, ll, dd = xs.shape
    pred = W.shape[0]
    wt = W.T
    b2 = b.reshape(1, pred)
    grid = (bb // _BBLK,)
    return pl.pallas_call(
        _head_kernel,
        grid=grid,
        in_specs=[
            pl.BlockSpec(memory_space=pl.ANY),
            pl.BlockSpec((_BBLK, ps), lambda t: (t, 0)),
            pl.BlockSpec((dd, pred), lambda t: (0, 0)),
            pl.BlockSpec((1, pred), lambda t: (0, 0)),
        ],
        out_specs=pl.BlockSpec((_BBLK, pred, cc), lambda t: (t, 0, 0)),
        out_shape=jax.ShapeDtypeStruct((bb, pred, cc), jnp.float32),
        scratch_shapes=[
            pltpu.VMEM((2, ps, _BBLK, cc, dd), jnp.float32),
            pltpu.SemaphoreType.DMA((2, ps)),
        ],
    )(xs, gates, wt, b2)
